# baseline (speedup 1.0000x reference)
import functools

import jax
import jax.numpy as jnp
from jax import lax
from jax.experimental import pallas as pl
from jax.experimental.pallas import tpu as pltpu
from jax.experimental.pallas import tpu_sc as plsc

B = 4
A = 8649
AP = 8704
N = 6000
NO = 1500
NOP = 1504
L = 16
NCHUNK = N // L
IOU_T = 0.7
V0, V1, V2, V3 = 0.1, 0.1, 0.2, 0.2

SR = 128
SCOL = 128
SORTN = SR * SCOL

_mesh = plsc.VectorSubcoreMesh(
    core_axis_name="c", subcore_axis_name="s", num_cores=2, num_subcores=16)


def _sort_body(prb_ref, skey_ref, sidx_ref):
    key = prb_ref[...]
    I = (lax.broadcasted_iota(jnp.int32, (B, SR, SCOL), 1) * SCOL
         + lax.broadcasted_iota(jnp.int32, (B, SR, SCOL), 2))
    idx = I
    for p in range(14):
        for q in range(p, -1, -1):
            s = 1 << q
            if s < SCOL:
                axis, sh = 2, s
            else:
                axis, sh = 1, s // SCOL
            ok_m = jnp.roll(key, -sh, axis=axis)
            ok_p = jnp.roll(key, sh, axis=axis)
            oi_m = jnp.roll(idx, -sh, axis=axis)
            oi_p = jnp.roll(idx, sh, axis=axis)
            up = (I & s) == 0
            ok = jnp.where(up, ok_m, ok_p)
            oi = jnp.where(up, oi_m, oi_p)
            desc = ((I >> (p + 1)) & 1) == 0
            sg = (key > ok) | ((key == ok) & (idx < oi))
            pref = sg ^ (up == desc)
            key = jnp.where(pref, ok, key)
            idx = jnp.where(pref, oi, idx)
    skey_ref[...] = key
    sidx_ref[...] = idx


_tc_sort = pl.pallas_call(
    _sort_body,
    out_shape=(jax.ShapeDtypeStruct((B, SR, SCOL), jnp.float32),
               jax.ShapeDtypeStruct((B, SR, SCOL), jnp.int32)),
)


NST = 16
SW = 384
NP = NST * SW
SCH = SW // L
TPI = 8
NPCH = NP // L


@functools.partial(
    pl.kernel,
    out_type=(
        jax.ShapeDtypeStruct((B, N), jnp.float32),
        jax.ShapeDtypeStruct((B, NOP), jnp.float32),
    ),
    mesh=_mesh,
    compiler_params=pltpu.CompilerParams(needs_layout_passes=False),
    scratch_types=[
        pltpu.VMEM((AP,), jnp.float32),
        pltpu.VMEM((AP,), jnp.float32),
        pltpu.VMEM((AP,), jnp.float32),
        pltpu.VMEM((AP,), jnp.float32),
        pltpu.VMEM((AP,), jnp.float32),
        pltpu.VMEM((AP,), jnp.float32),
        pltpu.VMEM((AP,), jnp.float32),
        pltpu.VMEM((AP,), jnp.float32),
        pltpu.VMEM((NP,), jnp.int32),
        pltpu.VMEM((NP,), jnp.float32),
        pltpu.VMEM((NP,), jnp.float32),
        pltpu.VMEM((NP,), jnp.float32),
        pltpu.VMEM((NP,), jnp.float32),
        pltpu.VMEM((NP,), jnp.float32),
        pltpu.VMEM((NP,), jnp.float32),
        pltpu.VMEM((2 * SW,), jnp.int32),
        pltpu.VMEM((SW,), jnp.int32),
        pltpu.VMEM((SW,), jnp.int32),
        pltpu.VMEM((L,), jnp.int32),
        pltpu.VMEM((N,), jnp.float32),
        pltpu.VMEM((NOP,), jnp.float32),
        pltpu.VMEM_SHARED((B * NST * SW,), jnp.int32),
        pltpu.VMEM_SHARED((B * NST * SW,), jnp.int32),
        pltpu.VMEM_SHARED((B * NST * L,), jnp.int32),
    ],
)
def _sc_nms(dts_hbm, anc_hbm, sc_hbm, idx_hbm, rois_hbm, rsc_hbm,
            d0v, d1v, d2v, d3v, a0v, a1v, a2v, a3v,
            iv, sv, y1v, x1v, y2v, x2v, arv, kov, pbv, pfv, cbv, obv, osv,
            spm, spl, spc):
    c = lax.axis_index("c")
    sid = lax.axis_index("s")
    b = c * 2 + sid // TPI
    t = sid % TPI
    lane = lax.iota(jnp.int32, L)

    pltpu.sync_copy(dts_hbm.at[b * 4], d0v)
    pltpu.sync_copy(dts_hbm.at[b * 4 + 1], d1v)
    pltpu.sync_copy(dts_hbm.at[b * 4 + 2], d2v)
    pltpu.sync_copy(dts_hbm.at[b * 4 + 3], d3v)
    pltpu.sync_copy(anc_hbm.at[0], a0v)
    pltpu.sync_copy(anc_hbm.at[1], a1v)
    pltpu.sync_copy(anc_hbm.at[2], a2v)
    pltpu.sync_copy(anc_hbm.at[3], a3v)
    pltpu.sync_copy(idx_hbm.at[b], iv)

    @pl.when(t == 0)
    def _ldsc():
        pltpu.sync_copy(sc_hbm.at[b], sv)

    def gather_body(ci, carry):
        s = pl.ds(ci * L, L)
        ix = iv[s]
        d0 = plsc.load_gather(d0v, [ix])
        d1 = plsc.load_gather(d1v, [ix])
        d2 = plsc.load_gather(d2v, [ix])
        d3 = plsc.load_gather(d3v, [ix])
        ay1 = plsc.load_gather(a0v, [ix])
        ax1 = plsc.load_gather(a1v, [ix])
        ay2 = plsc.load_gather(a2v, [ix])
        ax2 = plsc.load_gather(a3v, [ix])
        aw = ax2 - ax1
        ah = ay2 - ay1
        acx = ax1 + 0.5 * aw
        acy = ay1 + 0.5 * ah
        bw = jnp.exp(d3 * V3) * aw
        bh = jnp.exp(d2 * V2) * ah
        bcx = d1 * V1 * aw + acx
        bcy = d0 * V0 * ah + acy
        yy1 = jnp.clip(bcy - 0.5 * bh, 0.0, 1.0)
        xx1 = jnp.clip(bcx - 0.5 * bw, 0.0, 1.0)
        yy2 = jnp.clip(bh + (bcy - 0.5 * bh), 0.0, 1.0)
        xx2 = jnp.clip(bw + (bcx - 0.5 * bw), 0.0, 1.0)
        y1v[s] = yy1
        x1v[s] = xx1
        y2v[s] = yy2
        x2v[s] = xx2
        arv[s] = (yy2 - yy1) * (xx2 - xx1)
        return carry

    lax.fori_loop(0, NPCH, gather_body, 0)

    def kinit_body(u, carry):
        g0 = t * SW + u * L + lane
        kov[pl.ds(u * L, L)] = jnp.where(g0 < N, 0, 1)
        g1 = (15 - t) * SW + u * L + lane
        kov[pl.ds(SW + u * L, L)] = jnp.where(g1 < N, 0, 1)
        return carry

    lax.fori_loop(0, SCH, kinit_body, 0)

    def iou_supp(gbase, by1, bx1, by2, bx2, bar):
        s = pl.ds(gbase, L)
        iy1 = jnp.maximum(y1v[s], by1)
        ix1 = jnp.maximum(x1v[s], bx1)
        iy2 = jnp.minimum(y2v[s], by2)
        ix2 = jnp.minimum(x2v[s], bx2)
        inter = jnp.maximum(iy2 - iy1, 0.0) * jnp.maximum(ix2 - ix1, 0.0)
        denom = arv[s] + bar - inter + 1e-9
        return inter > IOU_T * denom

    def resolve_publish(kk, hh):
        def row_body(r, carry2):
            spr = jnp.full((L,), hh + r, jnp.int32)
            alive = plsc.load_gather(kov, [spr])

            @pl.when(alive[0] == 0)
            def _row():
                gi = kk * SW + r
                spg = jnp.full((L,), gi, jnp.int32)
                by1 = plsc.load_gather(y1v, [spg])
                bx1 = plsc.load_gather(x1v, [spg])
                by2 = plsc.load_gather(y2v, [spg])
                bx2 = plsc.load_gather(x2v, [spg])
                bar = plsc.load_gather(arv, [spg])

                ud = r // L
                supp0 = iou_supp(kk * SW + ud * L, by1, bx1, by2, bx2, bar)
                supp0 = supp0 & (ud * L + lane > r)
                plsc.addupdate(kov.at[pl.ds(hh + ud * L, L)],
                               jnp.where(supp0, 1, 0))

                @plsc.parallel_loop(ud + 1, SCH, unroll=2)
                def chunk_body(u):
                    supp = iou_supp(kk * SW + u * L, by1, bx1, by2, bx2, bar)
                    plsc.addupdate(kov.at[pl.ds(hh + u * L, L)],
                                   jnp.where(supp, 1, 0))
            return carry2

        lax.fori_loop(0, SW, row_body, 0)
        pltpu.sync_copy(kov.at[pl.ds(hh, SW)],
                        spm.at[pl.ds((b * NST + kk) * SW, SW)])

        def lcomp_body(u, base):
            kb = kov[pl.ds(hh + u * L, L)] == 0
            kbi = jnp.where(kb, 1, 0)
            slot = base + plsc.cumsum(kbi) - 1
            plsc.store_scatter(pbv, [slot], kk * SW + u * L + lane, mask=kb)
            return base + jnp.sum(kbi)

        cnt = lax.fori_loop(0, SCH, lcomp_body, 0)
        pltpu.sync_copy(pbv, spl.at[pl.ds((b * NST + kk) * SW, SW)])
        cbv[pl.ds(0, L)] = jnp.full((L,), cnt, jnp.int32)
        pltpu.sync_copy(cbv, spc.at[pl.ds((b * NST + kk) * L, L)])

    @pl.when(t == 0)
    def _prologue():
        resolve_publish(0, 0)

    def round_body(k, carry):
        plsc.subcore_barrier()

        @pl.when((t > k) | (15 - t > k))
        def _fetch_apply():
            pltpu.sync_copy(spl.at[pl.ds((b * NST + k) * SW, SW)], pfv)
            pltpu.sync_copy(spc.at[pl.ds((b * NST + k) * L, L)], cbv)
            cnt = cbv[pl.ds(0, L)][0]

            def apply_half(hh, sg):
                @plsc.parallel_loop(0, cnt)
                def row_body(rr):
                    spg = plsc.load_gather(pfv,
                                           [jnp.full((L,), rr, jnp.int32)])
                    by1 = plsc.load_gather(y1v, [spg])
                    bx1 = plsc.load_gather(x1v, [spg])
                    by2 = plsc.load_gather(y2v, [spg])
                    bx2 = plsc.load_gather(x2v, [spg])
                    bar = plsc.load_gather(arv, [spg])

                    @plsc.parallel_loop(0, SCH, unroll=2)
                    def chunk_body(u):
                        supp = iou_supp(sg * SW + u * L,
                                        by1, bx1, by2, bx2, bar)
                        plsc.addupdate(kov.at[pl.ds(hh + u * L, L)],
                                       jnp.where(supp, 1, 0))

            is_next0 = (k <= 6) & (t == k + 1)
            is_next1 = (k >= 7) & (t == 14 - k)

            @pl.when(is_next0)
            def _n0():
                apply_half(0, t)
                resolve_publish(k + 1, 0)
                apply_half(SW, 15 - t)

            @pl.when(is_next1)
            def _n1():
                apply_half(SW, 15 - t)
                resolve_publish(k + 1, SW)

            @pl.when(jnp.logical_not(is_next0 | is_next1))
            def _gen():
                @pl.when(t > k)
                def _h0():
                    apply_half(0, t)

                @pl.when(15 - t > k)
                def _h1():
                    apply_half(SW, 15 - t)
        return carry

    lax.fori_loop(0, NST - 1, round_body, 0)
    plsc.subcore_barrier()

    @pl.when(t == 0)
    def _compact():
        def zero_body(ci, carry):
            obv[pl.ds(ci * L, L)] = jnp.zeros((L,), jnp.float32)
            return carry

        lax.fori_loop(0, N // L, zero_body, 0)

        def zero2_body(ci, carry):
            osv[pl.ds(ci * L, L)] = jnp.zeros((L,), jnp.float32)
            return carry

        lax.fori_loop(0, NOP // L, zero2_body, 0)

        def stripe_body(k, base0):
            pltpu.sync_copy(spm.at[pl.ds((b * NST + k) * SW, SW)], pbv)

            def comp_body(u, base):
                sl = pl.ds(u * L, L)
                gsl = pl.ds(k * SW + u * L, L)
                kb = pbv[sl] == 0
                kbi = jnp.where(kb, 1, 0)
                pos = plsc.cumsum(kbi)
                slot = base + pos - 1
                wm = kb & (slot < NO)
                scs = sv[gsl]
                validf = jnp.where(scs > 0.0, 1.0, 0.0)
                plsc.store_scatter(osv, [slot], scs * validf, mask=wm)
                s4 = slot * 4
                plsc.store_scatter(obv, [s4], y1v[gsl] * validf, mask=wm)
                plsc.store_scatter(obv, [s4 + 1], x1v[gsl] * validf, mask=wm)
                plsc.store_scatter(obv, [s4 + 2], y2v[gsl] * validf, mask=wm)
                plsc.store_scatter(obv, [s4 + 3], x2v[gsl] * validf, mask=wm)
                return base + jnp.sum(kbi)

            return lax.fori_loop(0, SCH, comp_body, base0)

        lax.fori_loop(0, NST, stripe_body, 0)
        pltpu.sync_copy(obv, rois_hbm.at[b])
        pltpu.sync_copy(osv, rsc_hbm.at[b])


def kernel(rpn_bbox_deltas, rpn_probs, anchors):
    deltas = rpn_bbox_deltas.reshape(B, A, 4)
    probs = rpn_probs.reshape(B, A)
    dts = jnp.transpose(deltas, (0, 2, 1))
    dts = jnp.pad(dts, ((0, 0), (0, 0), (0, AP - A))).reshape(B * 4, AP)
    anc = jnp.pad(anchors.T, ((0, 0), (0, AP - A)))
    prbp = jnp.pad(probs, ((0, 0), (0, SORTN - A)), constant_values=-1.0)
    skey, sidx = _tc_sort(prbp.reshape(B, SR, SCOL))
    ps = jnp.pad(skey.reshape(B, SORTN)[:, :N], ((0, 0), (0, NP - N)))
    pi = jnp.pad(sidx.reshape(B, SORTN)[:, :N], ((0, 0), (0, NP - N)),
                 constant_values=A + 1)
    rois, rsc = _sc_nms(dts, anc, ps, pi)
    return rois.reshape(B, NO, 4), rsc[:, :NO]


# speedup vs baseline: 1.0239x; 1.0239x over previous
import functools

import jax
import jax.numpy as jnp
from jax import lax
from jax.experimental import pallas as pl
from jax.experimental.pallas import tpu as pltpu
from jax.experimental.pallas import tpu_sc as plsc

B = 4
A = 8649
AP = 8704
N = 6000
NO = 1500
NOP = 1504
L = 16
NCHUNK = N // L
IOU_T = 0.7
V0, V1, V2, V3 = 0.1, 0.1, 0.2, 0.2

SR = 128
SCOL = 128
SORTN = SR * SCOL

_mesh = plsc.VectorSubcoreMesh(
    core_axis_name="c", subcore_axis_name="s", num_cores=2, num_subcores=16)


def _sort_body(prb_ref, skey_ref, sidx_ref):
    key = prb_ref[...]
    I = (lax.broadcasted_iota(jnp.int32, (B, SR, SCOL), 1) * SCOL
         + lax.broadcasted_iota(jnp.int32, (B, SR, SCOL), 2))
    idx = I
    for p in range(14):
        for q in range(p, -1, -1):
            s = 1 << q
            if s < SCOL:
                axis, sh = 2, s
            else:
                axis, sh = 1, s // SCOL
            ok_m = jnp.roll(key, -sh, axis=axis)
            ok_p = jnp.roll(key, sh, axis=axis)
            oi_m = jnp.roll(idx, -sh, axis=axis)
            oi_p = jnp.roll(idx, sh, axis=axis)
            up = (I & s) == 0
            ok = jnp.where(up, ok_m, ok_p)
            oi = jnp.where(up, oi_m, oi_p)
            desc = ((I >> (p + 1)) & 1) == 0
            sg = (key > ok) | ((key == ok) & (idx < oi))
            pref = sg ^ (up == desc)
            key = jnp.where(pref, ok, key)
            idx = jnp.where(pref, oi, idx)
    skey_ref[...] = key
    sidx_ref[...] = idx


_tc_sort = pl.pallas_call(
    _sort_body,
    out_shape=(jax.ShapeDtypeStruct((B, SR, SCOL), jnp.float32),
               jax.ShapeDtypeStruct((B, SR, SCOL), jnp.int32)),
)


NST = 16
SW = 384
NP = NST * SW
SCH = SW // L
TPI = 8
NPCH = NP // L


@functools.partial(
    pl.kernel,
    out_type=(
        jax.ShapeDtypeStruct((B, N), jnp.float32),
        jax.ShapeDtypeStruct((B, NOP), jnp.float32),
    ),
    mesh=_mesh,
    compiler_params=pltpu.CompilerParams(needs_layout_passes=False),
    scratch_types=[
        pltpu.VMEM((4 * AP,), jnp.float32),
        pltpu.VMEM((4 * AP,), jnp.float32),
        pltpu.VMEM((NP,), jnp.int32),
        pltpu.VMEM((NP,), jnp.float32),
        pltpu.VMEM((NP,), jnp.float32),
        pltpu.VMEM((NP,), jnp.float32),
        pltpu.VMEM((NP,), jnp.float32),
        pltpu.VMEM((NP,), jnp.float32),
        pltpu.VMEM((NP,), jnp.float32),
        pltpu.VMEM((2 * SW,), jnp.int32),
        pltpu.VMEM((SW,), jnp.int32),
        pltpu.VMEM((SW,), jnp.int32),
        pltpu.VMEM((L,), jnp.int32),
        pltpu.VMEM((N,), jnp.float32),
        pltpu.VMEM((NOP,), jnp.float32),
        pltpu.VMEM_SHARED((B * NST * SW,), jnp.int32),
        pltpu.VMEM_SHARED((B * NST * SW,), jnp.int32),
        pltpu.VMEM_SHARED((B * NST * L,), jnp.int32),
    ],
)
def _sc_nms(dts_hbm, anc_hbm, sc_hbm, idx_hbm, rois_hbm, rsc_hbm,
            dfv, afv,
            iv, sv, y1v, x1v, y2v, x2v, arv, kov, pbv, pfv, cbv, obv, osv,
            spm, spl, spc):
    c = lax.axis_index("c")
    sid = lax.axis_index("s")
    b = c * 2 + sid // TPI
    t = sid % TPI
    lane = lax.iota(jnp.int32, L)

    pltpu.sync_copy(dts_hbm.at[b], dfv)
    pltpu.sync_copy(anc_hbm, afv)
    pltpu.sync_copy(idx_hbm.at[b], iv)

    @pl.when(t == 0)
    def _ldsc():
        pltpu.sync_copy(sc_hbm.at[b], sv)

    def gather_body(ci, carry):
        s = pl.ds(ci * L, L)
        ix4 = iv[s] * 4
        d0 = plsc.load_gather(dfv, [ix4])
        d1 = plsc.load_gather(dfv, [ix4 + 1])
        d2 = plsc.load_gather(dfv, [ix4 + 2])
        d3 = plsc.load_gather(dfv, [ix4 + 3])
        ay1 = plsc.load_gather(afv, [ix4])
        ax1 = plsc.load_gather(afv, [ix4 + 1])
        ay2 = plsc.load_gather(afv, [ix4 + 2])
        ax2 = plsc.load_gather(afv, [ix4 + 3])
        aw = ax2 - ax1
        ah = ay2 - ay1
        acx = ax1 + 0.5 * aw
        acy = ay1 + 0.5 * ah
        bw = jnp.exp(d3 * V3) * aw
        bh = jnp.exp(d2 * V2) * ah
        bcx = d1 * V1 * aw + acx
        bcy = d0 * V0 * ah + acy
        yy1 = jnp.clip(bcy - 0.5 * bh, 0.0, 1.0)
        xx1 = jnp.clip(bcx - 0.5 * bw, 0.0, 1.0)
        yy2 = jnp.clip(bh + (bcy - 0.5 * bh), 0.0, 1.0)
        xx2 = jnp.clip(bw + (bcx - 0.5 * bw), 0.0, 1.0)
        y1v[s] = yy1
        x1v[s] = xx1
        y2v[s] = yy2
        x2v[s] = xx2
        arv[s] = (yy2 - yy1) * (xx2 - xx1)
        return carry

    lax.fori_loop(0, NPCH, gather_body, 0)

    def kinit_body(u, carry):
        g0 = t * SW + u * L + lane
        kov[pl.ds(u * L, L)] = jnp.where(g0 < N, 0, 1)
        g1 = (15 - t) * SW + u * L + lane
        kov[pl.ds(SW + u * L, L)] = jnp.where(g1 < N, 0, 1)
        return carry

    lax.fori_loop(0, SCH, kinit_body, 0)

    def iou_supp(gbase, by1, bx1, by2, bx2, bar):
        s = pl.ds(gbase, L)
        iy1 = jnp.maximum(y1v[s], by1)
        ix1 = jnp.maximum(x1v[s], bx1)
        iy2 = jnp.minimum(y2v[s], by2)
        ix2 = jnp.minimum(x2v[s], bx2)
        inter = jnp.maximum(iy2 - iy1, 0.0) * jnp.maximum(ix2 - ix1, 0.0)
        denom = arv[s] + bar - inter + 1e-9
        return inter > IOU_T * denom

    def resolve_publish(kk, hh):
        def row_body(r, carry2):
            spr = jnp.full((L,), hh + r, jnp.int32)
            alive = plsc.load_gather(kov, [spr])

            @pl.when(alive[0] == 0)
            def _row():
                gi = kk * SW + r
                spg = jnp.full((L,), gi, jnp.int32)
                by1 = plsc.load_gather(y1v, [spg])
                bx1 = plsc.load_gather(x1v, [spg])
                by2 = plsc.load_gather(y2v, [spg])
                bx2 = plsc.load_gather(x2v, [spg])
                bar = plsc.load_gather(arv, [spg])

                ud = r // L
                supp0 = iou_supp(kk * SW + ud * L, by1, bx1, by2, bx2, bar)
                supp0 = supp0 & (ud * L + lane > r)
                plsc.addupdate(kov.at[pl.ds(hh + ud * L, L)],
                               jnp.where(supp0, 1, 0))

                @plsc.parallel_loop(ud + 1, SCH, unroll=2)
                def chunk_body(u):
                    supp = iou_supp(kk * SW + u * L, by1, bx1, by2, bx2, bar)
                    plsc.addupdate(kov.at[pl.ds(hh + u * L, L)],
                                   jnp.where(supp, 1, 0))
            return carry2

        lax.fori_loop(0, SW, row_body, 0)
        pltpu.sync_copy(kov.at[pl.ds(hh, SW)],
                        spm.at[pl.ds((b * NST + kk) * SW, SW)])

        def lcomp_body(u, base):
            kb = kov[pl.ds(hh + u * L, L)] == 0
            kbi = jnp.where(kb, 1, 0)
            slot = base + plsc.cumsum(kbi) - 1
            plsc.store_scatter(pbv, [slot], kk * SW + u * L + lane, mask=kb)
            return base + jnp.sum(kbi)

        cnt = lax.fori_loop(0, SCH, lcomp_body, 0)
        pltpu.sync_copy(pbv, spl.at[pl.ds((b * NST + kk) * SW, SW)])
        cbv[pl.ds(0, L)] = jnp.full((L,), cnt, jnp.int32)
        pltpu.sync_copy(cbv, spc.at[pl.ds((b * NST + kk) * L, L)])

    @pl.when(t == 0)
    def _prologue():
        resolve_publish(0, 0)

    def round_body(k, carry):
        plsc.subcore_barrier()

        @pl.when((t > k) | (15 - t > k))
        def _fetch_apply():
            pltpu.sync_copy(spl.at[pl.ds((b * NST + k) * SW, SW)], pfv)
            pltpu.sync_copy(spc.at[pl.ds((b * NST + k) * L, L)], cbv)
            cnt = cbv[pl.ds(0, L)][0]

            def apply_half(hh, sg):
                @plsc.parallel_loop(0, cnt)
                def row_body(rr):
                    spg = plsc.load_gather(pfv,
                                           [jnp.full((L,), rr, jnp.int32)])
                    by1 = plsc.load_gather(y1v, [spg])
                    bx1 = plsc.load_gather(x1v, [spg])
                    by2 = plsc.load_gather(y2v, [spg])
                    bx2 = plsc.load_gather(x2v, [spg])
                    bar = plsc.load_gather(arv, [spg])

                    @plsc.parallel_loop(0, SCH, unroll=2)
                    def chunk_body(u):
                        supp = iou_supp(sg * SW + u * L,
                                        by1, bx1, by2, bx2, bar)
                        plsc.addupdate(kov.at[pl.ds(hh + u * L, L)],
                                       jnp.where(supp, 1, 0))

            is_next0 = (k <= 6) & (t == k + 1)
            is_next1 = (k >= 7) & (t == 14 - k)

            @pl.when(is_next0)
            def _n0():
                apply_half(0, t)
                resolve_publish(k + 1, 0)
                apply_half(SW, 15 - t)

            @pl.when(is_next1)
            def _n1():
                apply_half(SW, 15 - t)
                resolve_publish(k + 1, SW)

            @pl.when(jnp.logical_not(is_next0 | is_next1))
            def _gen():
                @pl.when(t > k)
                def _h0():
                    apply_half(0, t)

                @pl.when(15 - t > k)
                def _h1():
                    apply_half(SW, 15 - t)
        return carry

    lax.fori_loop(0, NST - 1, round_body, 0)
    plsc.subcore_barrier()

    @pl.when(t == 0)
    def _compact():
        def zero_body(ci, carry):
            obv[pl.ds(ci * L, L)] = jnp.zeros((L,), jnp.float32)
            return carry

        lax.fori_loop(0, N // L, zero_body, 0)

        def zero2_body(ci, carry):
            osv[pl.ds(ci * L, L)] = jnp.zeros((L,), jnp.float32)
            return carry

        lax.fori_loop(0, NOP // L, zero2_body, 0)

        def stripe_body(k, base0):
            pltpu.sync_copy(spm.at[pl.ds((b * NST + k) * SW, SW)], pbv)

            def comp_body(u, base):
                sl = pl.ds(u * L, L)
                gsl = pl.ds(k * SW + u * L, L)
                kb = pbv[sl] == 0
                kbi = jnp.where(kb, 1, 0)
                pos = plsc.cumsum(kbi)
                slot = base + pos - 1
                wm = kb & (slot < NO)
                scs = sv[gsl]
                validf = jnp.where(scs > 0.0, 1.0, 0.0)
                plsc.store_scatter(osv, [slot], scs * validf, mask=wm)
                s4 = slot * 4
                plsc.store_scatter(obv, [s4], y1v[gsl] * validf, mask=wm)
                plsc.store_scatter(obv, [s4 + 1], x1v[gsl] * validf, mask=wm)
                plsc.store_scatter(obv, [s4 + 2], y2v[gsl] * validf, mask=wm)
                plsc.store_scatter(obv, [s4 + 3], x2v[gsl] * validf, mask=wm)
                return base + jnp.sum(kbi)

            return lax.fori_loop(0, SCH, comp_body, base0)

        lax.fori_loop(0, NST, stripe_body, 0)
        pltpu.sync_copy(obv, rois_hbm.at[b])
        pltpu.sync_copy(osv, rsc_hbm.at[b])


def kernel(rpn_bbox_deltas, rpn_probs, anchors):
    probs = rpn_probs.reshape(B, A)
    dts = jnp.pad(rpn_bbox_deltas.reshape(B, A * 4),
                  ((0, 0), (0, 4 * (AP - A))))
    anc = jnp.pad(anchors.reshape(A * 4), (0, 4 * (AP - A)))
    prbp = jnp.pad(probs, ((0, 0), (0, SORTN - A)), constant_values=-1.0)
    skey, sidx = _tc_sort(prbp.reshape(B, SR, SCOL))
    ps = jnp.pad(skey.reshape(B, SORTN)[:, :N], ((0, 0), (0, NP - N)))
    pi = jnp.pad(sidx.reshape(B, SORTN)[:, :N], ((0, 0), (0, NP - N)),
                 constant_values=A + 1)
    rois, rsc = _sc_nms(dts, anc, ps, pi)
    return rois.reshape(B, NO, 4), rsc[:, :NO]
